# baseline (device time: 100884 ns/iter reference)
import jax
import jax.numpy as jnp
from jax import lax
from jax.experimental import pallas as pl
from jax.experimental.pallas import tpu as pltpu

N_DEV = 8
M = 4096
D = 512
CHUNK = M // N_DEV


def kernel(partial, gamma):
    x = partial.reshape(M, D)
    g = gamma.reshape(1, D)

    def body(x_ref, g_ref, out_ref, comm_ref, send_sems, recv_sems):
        my = lax.axis_index("i")
        left = lax.rem(my + N_DEV - 1, N_DEV)
        right = lax.rem(my + 1, N_DEV)

        barrier_sem = pltpu.get_barrier_semaphore()
        for nbr in (left, right):
            pl.semaphore_signal(
                barrier_sem, inc=1,
                device_id=(nbr,), device_id_type=pl.DeviceIdType.MESH,
            )
        pl.semaphore_wait(barrier_sem, 2)

        first = lax.rem(my + N_DEV - 1, N_DEV)
        comm_ref[0, :, :] = x_ref[pl.ds(first * CHUNK, CHUNK), :]

        for s in range(N_DEV - 1):
            rdma = pltpu.make_async_remote_copy(
                src_ref=comm_ref.at[s],
                dst_ref=comm_ref.at[s + 1],
                send_sem=send_sems.at[s],
                recv_sem=recv_sems.at[s],
                device_id=(right,),
                device_id_type=pl.DeviceIdType.MESH,
            )
            rdma.start()
            rdma.wait()
            c = lax.rem(my + 2 * N_DEV - 2 - s, N_DEV)
            comm_ref[s + 1, :, :] += x_ref[pl.ds(c * CHUNK, CHUNK), :]

        y = comm_ref[N_DEV - 1, :, :]
        rms = jnp.sqrt(jnp.mean(y * y, axis=-1, keepdims=True) + 1e-6)
        out_ref[:, :] = y / rms * g_ref[0, :]

    return pl.pallas_call(
        body,
        out_shape=jax.ShapeDtypeStruct((CHUNK, D), jnp.float32),
        in_specs=[
            pl.BlockSpec(memory_space=pltpu.VMEM),
            pl.BlockSpec(memory_space=pltpu.VMEM),
        ],
        out_specs=pl.BlockSpec(memory_space=pltpu.VMEM),
        scratch_shapes=[
            pltpu.VMEM((N_DEV, CHUNK, D), jnp.float32),
            pltpu.SemaphoreType.DMA((N_DEV - 1,)),
            pltpu.SemaphoreType.DMA((N_DEV - 1,)),
        ],
        compiler_params=pltpu.CompilerParams(collective_id=0),
    )(x, g)


# device time: 64457 ns/iter; 1.5651x vs baseline; 1.5651x over previous
import jax
import jax.numpy as jnp
from jax import lax
from jax.experimental import pallas as pl
from jax.experimental.pallas import tpu as pltpu

N_DEV = 8
M = 4096
D = 512
CHUNK = M // N_DEV
H = D // 2


def kernel(partial, gamma):
    x = partial.reshape(M, D)
    g = gamma.reshape(1, D)

    def body(x_ref, g_ref, out_ref, commf_ref, commb_ref,
             sendf_sems, recvf_sems, sendb_sems, recvb_sems):
        my = lax.axis_index("i")
        left = lax.rem(my + N_DEV - 1, N_DEV)
        right = lax.rem(my + 1, N_DEV)

        barrier_sem = pltpu.get_barrier_semaphore()
        for nbr in (left, right):
            pl.semaphore_signal(
                barrier_sem, inc=1,
                device_id=(nbr,), device_id_type=pl.DeviceIdType.MESH,
            )
        pl.semaphore_wait(barrier_sem, 2)

        cf0 = lax.rem(my + N_DEV - 1, N_DEV)
        cb0 = lax.rem(my + 1, N_DEV)
        commf_ref[0, :, :] = x_ref[pl.ds(cf0 * CHUNK, CHUNK), 0:H]
        commb_ref[0, :, :] = x_ref[pl.ds(cb0 * CHUNK, CHUNK), H:D]

        for s in range(N_DEV - 1):
            rdma_f = pltpu.make_async_remote_copy(
                src_ref=commf_ref.at[s],
                dst_ref=commf_ref.at[s + 1],
                send_sem=sendf_sems.at[s],
                recv_sem=recvf_sems.at[s],
                device_id=(right,),
                device_id_type=pl.DeviceIdType.MESH,
            )
            rdma_b = pltpu.make_async_remote_copy(
                src_ref=commb_ref.at[s],
                dst_ref=commb_ref.at[s + 1],
                send_sem=sendb_sems.at[s],
                recv_sem=recvb_sems.at[s],
                device_id=(left,),
                device_id_type=pl.DeviceIdType.MESH,
            )
            rdma_f.start()
            rdma_b.start()
            rdma_f.wait()
            cf = lax.rem(my + 2 * N_DEV - 2 - s, N_DEV)
            commf_ref[s + 1, :, :] += x_ref[pl.ds(cf * CHUNK, CHUNK), 0:H]
            rdma_b.wait()
            cb = lax.rem(my + 2 + s, N_DEV)
            commb_ref[s + 1, :, :] += x_ref[pl.ds(cb * CHUNK, CHUNK), H:D]

        yf = commf_ref[N_DEV - 1, :, :]
        yb = commb_ref[N_DEV - 1, :, :]
        ssq = (jnp.sum(yf * yf, axis=-1, keepdims=True)
               + jnp.sum(yb * yb, axis=-1, keepdims=True))
        inv = lax.rsqrt(ssq / D + 1e-6)
        out_ref[:, 0:H] = yf * inv * g_ref[0, 0:H]
        out_ref[:, H:D] = yb * inv * g_ref[0, H:D]

    return pl.pallas_call(
        body,
        out_shape=jax.ShapeDtypeStruct((CHUNK, D), jnp.float32),
        in_specs=[
            pl.BlockSpec(memory_space=pltpu.VMEM),
            pl.BlockSpec(memory_space=pltpu.VMEM),
        ],
        out_specs=pl.BlockSpec(memory_space=pltpu.VMEM),
        scratch_shapes=[
            pltpu.VMEM((N_DEV, CHUNK, H), jnp.float32),
            pltpu.VMEM((N_DEV, CHUNK, H), jnp.float32),
            pltpu.SemaphoreType.DMA((N_DEV - 1,)),
            pltpu.SemaphoreType.DMA((N_DEV - 1,)),
            pltpu.SemaphoreType.DMA((N_DEV - 1,)),
            pltpu.SemaphoreType.DMA((N_DEV - 1,)),
        ],
        compiler_params=pltpu.CompilerParams(collective_id=0),
    )(x, g)


# device time: 51419 ns/iter; 1.9620x vs baseline; 1.2536x over previous
import jax
import jax.numpy as jnp
from jax import lax
from jax.experimental import pallas as pl
from jax.experimental.pallas import tpu as pltpu

N_DEV = 8
M = 4096
D = 512
CHUNK = M // N_DEV
Q = D // 4


def kernel(partial, gamma):
    x = partial.reshape(M, D)
    g = gamma.reshape(1, D)

    def body(x_ref, g_ref, out_ref,
             cf0, cf1, cb0, cb1,
             sf0, rf0, sf1, rf1, sb0, rb0, sb1, rb1):
        my = lax.axis_index("i")
        left = lax.rem(my + N_DEV - 1, N_DEV)
        right = lax.rem(my + 1, N_DEV)

        barrier_sem = pltpu.get_barrier_semaphore()
        for nbr in (left, right):
            pl.semaphore_signal(
                barrier_sem, inc=1,
                device_id=(nbr,), device_id_type=pl.DeviceIdType.MESH,
            )
        pl.semaphore_wait(barrier_sem, 2)

        rings = (
            (cf0, sf0, rf0, 0 * Q, right),
            (cb0, sb0, rb0, 2 * Q, left),
            (cf1, sf1, rf1, 1 * Q, right),
            (cb1, sb1, rb1, 3 * Q, left),
        )

        def acc_chunk(is_fwd, s):
            if is_fwd:
                return lax.rem(my + 2 * N_DEV - 2 - s, N_DEV)
            return lax.rem(my + 2 + s, N_DEV)

        cf_first = lax.rem(my + N_DEV - 1, N_DEV)
        cb_first = lax.rem(my + 1, N_DEV)
        for comm, _, _, lo, tgt in rings:
            first = cf_first if tgt is right else cb_first
            comm[0, :, :] = x_ref[pl.ds(first * CHUNK, CHUNK), lo:lo + Q]

        def mk(ring, s):
            comm, ssem, rsem, _, tgt = ring
            return pltpu.make_async_remote_copy(
                src_ref=comm.at[s],
                dst_ref=comm.at[s + 1],
                send_sem=ssem.at[s],
                recv_sem=rsem.at[s],
                device_id=(tgt,),
                device_id_type=pl.DeviceIdType.MESH,
            )

        descs = [[None] * (N_DEV - 1) for _ in rings]
        for r, ring in enumerate(rings):
            descs[r][0] = mk(ring, 0)
            descs[r][0].start()

        for s in range(N_DEV - 1):
            for r, ring in enumerate(rings):
                comm, _, _, lo, tgt = ring
                descs[r][s].wait_recv()
                c = acc_chunk(tgt is right, s)
                comm[s + 1, :, :] += x_ref[pl.ds(c * CHUNK, CHUNK), lo:lo + Q]
                if s + 1 < N_DEV - 1:
                    descs[r][s + 1] = mk(ring, s + 1)
                    descs[r][s + 1].start()

        for r in range(len(rings)):
            for s in range(N_DEV - 1):
                descs[r][s].wait_send()

        ys = [comm[N_DEV - 1, :, :] for comm, _, _, _, _ in rings]
        ssq = ys[0] * ys[0]
        for y in ys[1:]:
            ssq += y * y
        inv = lax.rsqrt(jnp.sum(ssq, axis=-1, keepdims=True) / D + 1e-6)
        for (comm, _, _, lo, _), y in zip(rings, ys):
            out_ref[:, lo:lo + Q] = y * inv * g_ref[0, lo:lo + Q]

    return pl.pallas_call(
        body,
        out_shape=jax.ShapeDtypeStruct((CHUNK, D), jnp.float32),
        in_specs=[
            pl.BlockSpec(memory_space=pltpu.VMEM),
            pl.BlockSpec(memory_space=pltpu.VMEM),
        ],
        out_specs=pl.BlockSpec(memory_space=pltpu.VMEM),
        scratch_shapes=[
            pltpu.VMEM((N_DEV, CHUNK, Q), jnp.float32),
            pltpu.VMEM((N_DEV, CHUNK, Q), jnp.float32),
            pltpu.VMEM((N_DEV, CHUNK, Q), jnp.float32),
            pltpu.VMEM((N_DEV, CHUNK, Q), jnp.float32),
        ] + [pltpu.SemaphoreType.DMA((N_DEV - 1,)) for _ in range(8)],
        compiler_params=pltpu.CompilerParams(collective_id=0),
    )(x, g)
